# Initial kernel scaffold; baseline (speedup 1.0000x reference)
import jax
import jax.numpy as jnp
from jax.experimental import pallas as pl


def kernel(combinations, weight):
    raise NotImplementedError("write your pallas kernel here")


# speedup vs baseline: 6.9068x; 6.9068x over previous
import functools

import jax
import jax.numpy as jnp
from jax import lax
from jax.experimental import pallas as pl
from jax.experimental.pallas import tpu as pltpu
from jax.experimental.pallas import tpu_sc as plsc

NUM_NODES = 100000
EMBED_DIM = 64
BATCH = 16384
COMB_LEN = 20

NC = 2
NS = 16
NW = NC * NS
B_PER_W = BATCH // NW
CB = 32
NCHUNK = B_PER_W // CB
ROWS_PER_CHUNK = CB * COMB_LEN
IDX_ROWS = ROWS_PER_CHUNK // 128


def _sc_body(comb_hbm, weight_hbm, part_hbm, idx_v, rows_v, outc_v, sem):
    wid = lax.axis_index("s") * NC + lax.axis_index("c")

    def chunk_body(c, _):
        i0 = wid * (NCHUNK * ROWS_PER_CHUNK) + c * ROWS_PER_CHUNK
        for j in range(IDX_ROWS):
            pltpu.sync_copy(comb_hbm.at[pl.ds(i0 + j * 128, 128)], idx_v.at[j])
        handles = [
            pltpu.async_copy(
                weight_hbm.at[idx_v.at[j]],
                rows_v.at[pl.ds(j * 128, 128)],
                sem,
            )
            for j in range(IDX_ROWS)
        ]
        for h in handles:
            h.wait()

        def elem_body(e, _):
            r0 = e * COMB_LEN
            acc = [rows_v[r0, pl.ds(k * 16, 16)] for k in range(4)]
            for l in range(1, COMB_LEN):
                for k in range(4):
                    acc[k] = acc[k] * rows_v[r0 + l, pl.ds(k * 16, 16)]
            s = (acc[0] + acc[1]) + (acc[2] + acc[3])
            outc_v[pl.ds(e * 16, 16)] = s
            return ()

        lax.fori_loop(0, CB, elem_body, ())
        pltpu.sync_copy(
            outc_v,
            part_hbm.at[pl.ds((wid * NCHUNK + c) * (CB * 16), CB * 16)],
        )
        return ()

    lax.fori_loop(0, NCHUNK, chunk_body, ())


def _tc_reduce_body(part_ref, out_ref):
    out_ref[...] = jnp.sum(part_ref[...], axis=1)


@jax.jit
def _hyper_embed(comb1d, weight):
    mesh = plsc.VectorSubcoreMesh(core_axis_name="c", subcore_axis_name="s")
    sc = functools.partial(
        pl.kernel,
        mesh=mesh,
        compiler_params=pltpu.CompilerParams(use_tc_tiling_on_sc=False),
        out_type=jax.ShapeDtypeStruct((BATCH * 16,), jnp.float32),
        scratch_types=[
            pltpu.VMEM((IDX_ROWS, 128), jnp.int32),
            pltpu.VMEM((ROWS_PER_CHUNK, EMBED_DIM), jnp.float32),
            pltpu.VMEM((CB * 16,), jnp.float32),
            pltpu.SemaphoreType.DMA,
        ],
    )(_sc_body)
    partials = sc(comb1d, weight)
    return pl.pallas_call(
        _tc_reduce_body,
        out_shape=jax.ShapeDtypeStruct((BATCH,), jnp.float32),
    )(partials.reshape(BATCH, 16))


def kernel(combinations, weight):
    comb1d = combinations.astype(jnp.int32).reshape(-1)
    return _hyper_embed(comb1d, weight)


# speedup vs baseline: 10.0980x; 1.4620x over previous
import functools

import jax
import jax.numpy as jnp
from jax import lax
from jax.experimental import pallas as pl
from jax.experimental.pallas import tpu as pltpu
from jax.experimental.pallas import tpu_sc as plsc

NUM_NODES = 100000
EMBED_DIM = 64
BATCH = 16384
COMB_LEN = 20

NC = 2
NS = 16
NW = NC * NS
B_PER_W = BATCH // NW
CB = 32
NCHUNK = B_PER_W // CB
ROWS_PER_CHUNK = CB * COMB_LEN
IDX_ROWS = ROWS_PER_CHUNK // 128
IDX_PER_W = B_PER_W * COMB_LEN


def _sc_body(comb_hbm, weight_hbm, part_hbm, idx_v, rows_v, outc_v,
             gsem0, gsem1, osem):
    wid = lax.axis_index("s") * NC + lax.axis_index("c")
    gsems = (gsem0, gsem1)

    pltpu.sync_copy(comb_hbm.at[pl.ds(wid * IDX_PER_W, IDX_PER_W)], idx_v)

    def fire(buf, sem, c):
        for j in range(IDX_ROWS):
            pltpu.async_copy(
                weight_hbm.at[idx_v.at[pl.ds(c * ROWS_PER_CHUNK + j * 128, 128)]],
                rows_v.at[buf, pl.ds(j * 128, 128)],
                sem,
            )

    def drain_rows(buf, sem):
        pltpu.make_async_copy(
            weight_hbm.at[pl.ds(0, ROWS_PER_CHUNK)], rows_v.at[buf], sem
        ).wait()

    def drain_out(buf):
        pltpu.make_async_copy(
            outc_v.at[buf], part_hbm.at[pl.ds(0, CB * 16)], osem
        ).wait()

    def compute(buf, c, need_drain):
        drain_rows(buf, gsems[buf])

        @pl.when(need_drain)
        def _():
            drain_out(buf)

        def elem_body(e, _):
            r0 = e * COMB_LEN
            acc = [rows_v[buf, r0, pl.ds(k * 16, 16)] for k in range(4)]
            for l in range(1, COMB_LEN):
                for k in range(4):
                    acc[k] = acc[k] * rows_v[buf, r0 + l, pl.ds(k * 16, 16)]
            s = (acc[0] + acc[1]) + (acc[2] + acc[3])
            outc_v[buf, pl.ds(e * 16, 16)] = s
            return ()

        lax.fori_loop(0, CB, elem_body, ())
        pltpu.async_copy(
            outc_v.at[buf],
            part_hbm.at[pl.ds((wid * NCHUNK + c) * (CB * 16), CB * 16)],
            osem,
        )

    fire(0, gsem0, 0)

    def pair_body(i, _):
        c0 = i * 2
        fire(1, gsem1, c0 + 1)
        compute(0, c0, i > 0)

        @pl.when(i < NCHUNK // 2 - 1)
        def _():
            fire(0, gsem0, c0 + 2)

        compute(1, c0 + 1, i > 0)
        return ()

    lax.fori_loop(0, NCHUNK // 2, pair_body, ())
    drain_out(0)
    drain_out(1)


def _tc_reduce_body(part_ref, out_ref):
    out_ref[...] = jnp.sum(part_ref[...], axis=1)


@jax.jit
def _hyper_embed(comb1d, weight):
    mesh = plsc.VectorSubcoreMesh(core_axis_name="c", subcore_axis_name="s")
    sc = functools.partial(
        pl.kernel,
        mesh=mesh,
        compiler_params=pltpu.CompilerParams(use_tc_tiling_on_sc=False),
        out_type=jax.ShapeDtypeStruct((BATCH * 16,), jnp.float32),
        scratch_types=[
            pltpu.VMEM((IDX_PER_W,), jnp.int32),
            pltpu.VMEM((2, ROWS_PER_CHUNK, EMBED_DIM), jnp.float32),
            pltpu.VMEM((2, CB * 16), jnp.float32),
            pltpu.SemaphoreType.DMA,
            pltpu.SemaphoreType.DMA,
            pltpu.SemaphoreType.DMA,
        ],
    )(_sc_body)
    partials = sc(comb1d, weight)
    return pl.pallas_call(
        _tc_reduce_body,
        out_shape=jax.ShapeDtypeStruct((BATCH,), jnp.float32),
    )(partials.reshape(BATCH, 16))


def kernel(combinations, weight):
    comb1d = combinations.astype(jnp.int32).reshape(-1)
    return _hyper_embed(comb1d, weight)
